# Initial kernel scaffold; baseline (speedup 1.0000x reference)
import jax
import jax.numpy as jnp
from jax.experimental import pallas as pl


def kernel(x, edge_index, edge_attr, W_i, W_f, W_c, W_o, b_i, b_f, b_c, b_o, w_ci, w_cf, w_co, T_i, T_f, T_c, T_o, cb_i, cb_f, cb_c, cb_o, W1, b1, W2, b2):
    raise NotImplementedError("write your pallas kernel here")


# speedup vs baseline: 2.3115x; 2.3115x over previous
import functools

import jax
import jax.numpy as jnp
from jax import lax
from jax.experimental import pallas as pl
from jax.experimental.pallas import tpu as pltpu
from jax.experimental.pallas import tpu_sc as plsc

_NC = 2
_NS = 16
_NW = _NC * _NS
_LANES = 16
_CH = 80


def _node_stage(x, W_i, W_c, W_o, bi, bc, bo, w_co, W1a, W1b):
    N, D = x.shape
    H = W_i.shape[1]
    TN = 1000
    grid = (N // TN,)

    def body(x_ref, wi_ref, wc_ref, wo_ref, bi_ref, bc_ref, bo_ref, wco_ref,
             w1a_ref, w1b_ref, a_ref, b_ref):
        xb = x_ref[...]
        gi = jax.nn.sigmoid(
            jnp.dot(xb, wi_ref[...], preferred_element_type=jnp.float32)
            + bi_ref[...])
        gc = jnp.tanh(
            jnp.dot(xb, wc_ref[...], preferred_element_type=jnp.float32)
            + bc_ref[...])
        c = gi * gc
        go = jax.nn.sigmoid(
            jnp.dot(xb, wo_ref[...], preferred_element_type=jnp.float32)
            + bo_ref[...] + wco_ref[...] * c)
        h = go * jnp.tanh(c)
        a_ref[...] = jnp.dot(h, w1a_ref[...], preferred_element_type=jnp.float32)
        b_ref[...] = jnp.dot(h, w1b_ref[...], preferred_element_type=jnp.float32)

    full = lambda s: pl.BlockSpec(s, lambda i: (0, 0))
    return pl.pallas_call(
        body,
        grid=grid,
        in_specs=[
            pl.BlockSpec((TN, D), lambda i: (i, 0)),
            full((D, H)), full((D, H)), full((D, H)),
            full((1, H)), full((1, H)), full((1, H)), full((1, H)),
            full((H, H)), full((H, H)),
        ],
        out_specs=[
            pl.BlockSpec((TN, H), lambda i: (i, 0)),
            pl.BlockSpec((TN, H), lambda i: (i, 0)),
        ],
        out_shape=[
            jax.ShapeDtypeStruct((N, H), jnp.float32),
            jax.ShapeDtypeStruct((N, H), jnp.float32),
        ],
    )(x, W_i, W_c, W_o, bi, bc, bo, w_co, W1a, W1b)


def _gather_add(A, B, src, dst):
    E = src.shape[0]
    H = A.shape[1]
    per_w = E // _NW
    n_chunks = per_w // _CH
    mesh = plsc.VectorSubcoreMesh(core_axis_name="c", subcore_axis_name="s")

    @functools.partial(
        pl.kernel,
        mesh=mesh,
        out_type=jax.ShapeDtypeStruct((E, H), jnp.float32),
        scratch_types=[
            pltpu.VMEM((_CH,), jnp.int32),
            pltpu.VMEM((_CH,), jnp.int32),
            pltpu.VMEM((_CH, H), jnp.float32),
            pltpu.VMEM((_CH, H), jnp.float32),
            pltpu.SemaphoreType.DMA,
            pltpu.SemaphoreType.DMA,
        ],
    )
    def k(a_hbm, b_hbm, src_hbm, dst_hbm, g_hbm, idx_s, idx_d, bufA, bufB,
          semA, semB):
        wid = lax.axis_index("s") * _NC + lax.axis_index("c")
        w_base = wid * per_w

        def chunk(t, carry):
            base = w_base + t * _CH
            pltpu.sync_copy(src_hbm.at[pl.ds(base, _CH)], idx_s)
            pltpu.sync_copy(dst_hbm.at[pl.ds(base, _CH)], idx_d)
            cA = pltpu.async_copy(a_hbm.at[idx_s], bufA, semA)
            cB = pltpu.async_copy(b_hbm.at[idx_d], bufB, semB)
            cA.wait()
            cB.wait()

            def row(i, c2):
                for j in range(H // _LANES):
                    sl = pl.ds(j * _LANES, _LANES)
                    bufA[i, sl] = bufA[i, sl] + bufB[i, sl]
                return c2

            lax.fori_loop(0, _CH, row, 0)
            pltpu.sync_copy(bufA, g_hbm.at[pl.ds(base, _CH)])
            return carry

        lax.fori_loop(0, n_chunks, chunk, 0)

    return k(A, B, src, dst)


def _edge_stage(G, edge_attr, W1e, b1, W2, b2):
    E, H = G.shape
    DE = edge_attr.shape[1]
    C = W2.shape[1]
    TE = 2000
    grid = (E // TE,)

    def body(g_ref, attr_ref, w1e_ref, b1_ref, w2_ref, b2_ref, out_ref):
        e = jnp.dot(attr_ref[...], w1e_ref[...],
                    preferred_element_type=jnp.float32)
        hid = jnp.maximum(g_ref[...] + e + b1_ref[...], 0.0)
        out_ref[...] = (
            jnp.dot(hid, w2_ref[...], preferred_element_type=jnp.float32)
            + b2_ref[...])

    full = lambda s: pl.BlockSpec(s, lambda i: (0, 0))
    return pl.pallas_call(
        body,
        grid=grid,
        in_specs=[
            pl.BlockSpec((TE, H), lambda i: (i, 0)),
            pl.BlockSpec((TE, DE), lambda i: (i, 0)),
            full((DE, H)), full((1, H)), full((H, C)), full((1, C)),
        ],
        out_specs=pl.BlockSpec((TE, C), lambda i: (i, 0)),
        out_shape=jax.ShapeDtypeStruct((E, C), jnp.float32),
    )(G, edge_attr, W1e, b1, W2, b2)


def kernel(x, edge_index, edge_attr, W_i, W_f, W_c, W_o, b_i, b_f, b_c, b_o,
           w_ci, w_cf, w_co, T_i, T_f, T_c, T_o, cb_i, cb_f, cb_c, cb_o,
           W1, b1, W2, b2):
    H = W_i.shape[1]
    bi = b_i + cb_i[None, :]
    bc = b_c + cb_c[None, :]
    bo = b_o + cb_o[None, :]
    W1a = W1[:H]
    W1b = W1[H:2 * H]
    W1e = W1[2 * H:]

    A, B = _node_stage(x, W_i, W_c, W_o, bi, bc, bo, w_co, W1a, W1b)
    G = _gather_add(A, B, edge_index[0], edge_index[1])
    return _edge_stage(G, edge_attr, W1e, b1[None, :], W2, b2[None, :])


# speedup vs baseline: 3.1960x; 1.3827x over previous
import functools

import jax
import jax.numpy as jnp
from jax import lax
from jax.experimental import pallas as pl
from jax.experimental.pallas import tpu as pltpu
from jax.experimental.pallas import tpu_sc as plsc

_NC = 2
_NS = 16
_NW = _NC * _NS
_LANES = 16
_CH = 80
_NBUF = 5
_HIMASK = -65536


def _node_stage(x, W_i, W_c, W_o, bi, bc, bo, w_co, W1a, W1b):
    N, D = x.shape
    H = W_i.shape[1]
    TN = 1000
    grid = (N // TN,)

    def body(x_ref, wi_ref, wc_ref, wo_ref, bi_ref, bc_ref, bo_ref, wco_ref,
             w1a_ref, w1b_ref, a_ref, b_ref):
        xb = x_ref[...]
        gi = jax.nn.sigmoid(
            jnp.dot(xb, wi_ref[...], preferred_element_type=jnp.float32)
            + bi_ref[...])
        gc = jnp.tanh(
            jnp.dot(xb, wc_ref[...], preferred_element_type=jnp.float32)
            + bc_ref[...])
        c = gi * gc
        go = jax.nn.sigmoid(
            jnp.dot(xb, wo_ref[...], preferred_element_type=jnp.float32)
            + bo_ref[...] + wco_ref[...] * c)
        h = go * jnp.tanh(c)
        a_ref[...] = jnp.dot(h, w1a_ref[...], preferred_element_type=jnp.float32)
        b_ref[...] = jnp.dot(h, w1b_ref[...], preferred_element_type=jnp.float32)

    full = lambda s: pl.BlockSpec(s, lambda i: (0, 0))
    return pl.pallas_call(
        body,
        grid=grid,
        in_specs=[
            pl.BlockSpec((TN, D), lambda i: (i, 0)),
            full((D, H)), full((D, H)), full((D, H)),
            full((1, H)), full((1, H)), full((1, H)), full((1, H)),
            full((H, H)), full((H, H)),
        ],
        out_specs=[
            pl.BlockSpec((TN, H), lambda i: (i, 0)),
            pl.BlockSpec((TN, H), lambda i: (i, 0)),
        ],
        out_shape=[
            jax.ShapeDtypeStruct((N, H), jnp.float32),
            jax.ShapeDtypeStruct((N, H), jnp.float32),
        ],
    )(x, W_i, W_c, W_o, bi, bc, bo, w_co, W1a, W1b)


def _pack_bf16_halves(t):
    W = t.shape[1] // 2
    tb = t.astype(jnp.bfloat16)
    lo = lax.bitcast_convert_type(tb[:, :W], jnp.uint16).astype(jnp.uint32)
    hi = lax.bitcast_convert_type(tb[:, W:], jnp.uint16).astype(jnp.uint32)
    return lax.bitcast_convert_type(lo | (hi << 16), jnp.int32)


def _packed_add(a, b):
    f32 = lambda v: lax.bitcast_convert_type(v, jnp.float32)
    i32 = lambda v: lax.bitcast_convert_type(v, jnp.int32)
    lo = i32(f32(a << 16) + f32(b << 16))
    hi = i32(f32(a & _HIMASK) + f32(b & _HIMASK))
    lo16 = lax.shift_right_logical(lo + 0x8000, 16)
    hi16 = (hi + 0x8000) & _HIMASK
    return lo16 | hi16


def _gather_add(A_pk, B_pk, src, dst):
    E = src.shape[0]
    HW = A_pk.shape[1]
    per_w = E // _NW
    n_chunks = per_w // _CH
    n_blocks = n_chunks // _NBUF
    src3 = src.reshape(_NW, n_chunks, _CH)
    dst3 = dst.reshape(_NW, n_chunks, _CH)
    mesh = plsc.VectorSubcoreMesh(core_axis_name="c", subcore_axis_name="s")

    scratch = [
        pltpu.VMEM((n_chunks, _CH), jnp.int32),
        pltpu.VMEM((n_chunks, _CH), jnp.int32),
    ]
    scratch += [pltpu.VMEM((_CH, HW), jnp.int32) for _ in range(3 * _NBUF)]
    scratch += [pltpu.SemaphoreType.DMA for _ in range(2 * _NBUF)]

    @functools.partial(
        pl.kernel,
        mesh=mesh,
        out_type=jax.ShapeDtypeStruct((E, HW), jnp.int32),
        scratch_types=scratch,
        compiler_params=pltpu.CompilerParams(use_tc_tiling_on_sc=False),
    )
    def k(a_hbm, b_hbm, src_hbm, dst_hbm, g_hbm, idx_s, idx_d, *scr):
        bufA = scr[0:_NBUF]
        bufB = scr[_NBUF:2 * _NBUF]
        bufO = scr[2 * _NBUF:3 * _NBUF]
        semg = scr[3 * _NBUF:4 * _NBUF]
        sems = scr[4 * _NBUF:5 * _NBUF]

        wid = lax.axis_index("s") * _NC + lax.axis_index("c")
        w_base = wid * per_w

        pltpu.sync_copy(src_hbm.at[wid], idx_s)
        pltpu.sync_copy(dst_hbm.at[wid], idx_d)

        def issue_gathers(t, b):
            pltpu.async_copy(a_hbm.at[idx_s.at[t]], bufA[b], semg[b])
            pltpu.async_copy(b_hbm.at[idx_d.at[t]], bufB[b], semg[b])

        for b in range(_NBUF):
            issue_gathers(b, b)

        def block(g, carry):
            for b in range(_NBUF):
                t = g * _NBUF + b
                pltpu.make_async_copy(
                    a_hbm.at[idx_s.at[0]], bufA[b], semg[b]).wait()
                pltpu.make_async_copy(
                    b_hbm.at[idx_d.at[0]], bufB[b], semg[b]).wait()

                @pl.when(g >= 1)
                def _():
                    pltpu.make_async_copy(
                        bufO[b], g_hbm.at[pl.ds(0, _CH)], sems[b]).wait()

                def row(i, c2):
                    for j in range(HW // _LANES):
                        sl = pl.ds(j * _LANES, _LANES)
                        bufO[b][i, sl] = _packed_add(
                            bufA[b][i, sl], bufB[b][i, sl])
                    return c2

                lax.fori_loop(0, _CH, row, 0)

                pltpu.async_copy(
                    bufO[b], g_hbm.at[pl.ds(w_base + t * _CH, _CH)], sems[b])

                @pl.when(g < n_blocks - 1)
                def _():
                    issue_gathers(t + _NBUF, b)
            return carry

        lax.fori_loop(0, n_blocks, block, 0)

        for b in range(_NBUF):
            pltpu.make_async_copy(
                bufO[b], g_hbm.at[pl.ds(0, _CH)], sems[b]).wait()

    return k(A_pk, B_pk, src3, dst3)


def _edge_stage(G_pk, edge_attr, W1e, b1, W2, b2):
    E, HW = G_pk.shape
    DE = edge_attr.shape[1]
    C = W2.shape[1]
    TE = 4000
    grid = (E // TE,)

    def body(g_ref, attr_ref, w1e_ref, b1_ref, w2_ref, b2_ref, out_ref):
        bits = g_ref[...]
        lo = lax.bitcast_convert_type(bits << 16, jnp.float32)
        hi = lax.bitcast_convert_type(bits & _HIMASK, jnp.float32)
        g = jnp.concatenate([lo, hi], axis=1)
        e = jnp.dot(attr_ref[...], w1e_ref[...],
                    preferred_element_type=jnp.float32)
        hid = jnp.maximum(g + e + b1_ref[...], 0.0)
        out_ref[...] = (
            jnp.dot(hid, w2_ref[...], preferred_element_type=jnp.float32)
            + b2_ref[...])

    full = lambda s: pl.BlockSpec(s, lambda i: (0, 0))
    return pl.pallas_call(
        body,
        grid=grid,
        in_specs=[
            pl.BlockSpec((TE, HW), lambda i: (i, 0)),
            pl.BlockSpec((TE, DE), lambda i: (i, 0)),
            full((DE, 2 * HW)), full((1, 2 * HW)), full((2 * HW, C)),
            full((1, C)),
        ],
        out_specs=pl.BlockSpec((TE, C), lambda i: (i, 0)),
        out_shape=jax.ShapeDtypeStruct((E, C), jnp.float32),
    )(G_pk, edge_attr, W1e, b1, W2, b2)


def kernel(x, edge_index, edge_attr, W_i, W_f, W_c, W_o, b_i, b_f, b_c, b_o,
           w_ci, w_cf, w_co, T_i, T_f, T_c, T_o, cb_i, cb_f, cb_c, cb_o,
           W1, b1, W2, b2):
    H = W_i.shape[1]
    bi = b_i + cb_i[None, :]
    bc = b_c + cb_c[None, :]
    bo = b_o + cb_o[None, :]
    W1a = W1[:H]
    W1b = W1[H:2 * H]
    W1e = W1[2 * H:]

    A, B = _node_stage(x, W_i, W_c, W_o, bi, bc, bo, w_co, W1a, W1b)
    A_pk = _pack_bf16_halves(A)
    B_pk = _pack_bf16_halves(B)
    G_pk = _gather_add(A_pk, B_pk, edge_index[0], edge_index[1])
    return _edge_stage(G_pk, edge_attr, W1e, b1[None, :], W2, b2[None, :])
